# baseline (speedup 1.0000x reference)
import functools

import jax
import jax.numpy as jnp
from jax import lax
from jax.experimental import pallas as pl
from jax.experimental.pallas import tpu as pltpu

NUM_EXPERTS = 64
TILE = 1024
NEG_INF = float("-inf")


def _router_kernel(x_ref, w_ref, b_ref, w1_ref, w2_ref, i1_ref, i2_ref):
    logits = lax.dot_general(
        x_ref[0], w_ref[...],
        dimension_numbers=(((1,), (1,)), ((), ())),
        preferred_element_type=jnp.float32,
    )
    logits = logits + b_ref[...]

    t = logits.shape[0]
    iota = lax.broadcasted_iota(jnp.int32, (t, NUM_EXPERTS), 1)
    big = jnp.int32(NUM_EXPERTS)

    m1 = jnp.max(logits, axis=1, keepdims=True)
    i1 = jnp.min(jnp.where(logits == m1, iota, big), axis=1, keepdims=True)
    masked = jnp.where(iota == i1, NEG_INF, logits)
    m2 = jnp.max(masked, axis=1, keepdims=True)
    i2 = jnp.min(jnp.where(masked == m2, iota, big), axis=1, keepdims=True)

    w1 = jax.nn.sigmoid(m1 - m2)
    w2 = 1.0 - w1

    w1_ref[...] = w1.reshape(1, 1, t)
    w2_ref[...] = w2.reshape(1, 1, t)
    i1_ref[...] = i1.reshape(1, 1, t)
    i2_ref[...] = i2.reshape(1, 1, t)


@functools.partial(jax.jit, static_argnames=())
def _run(x, W, b2d):
    bsz, seq, d = x.shape
    grid = (bsz, seq // TILE)
    flat = jax.ShapeDtypeStruct((bsz, 1, seq), jnp.float32)
    flati = jax.ShapeDtypeStruct((bsz, 1, seq), jnp.int32)
    ospec = pl.BlockSpec((1, 1, TILE), lambda bi, i: (bi, 0, i))
    w1, w2, i1, i2 = pl.pallas_call(
        _router_kernel,
        grid=grid,
        in_specs=[
            pl.BlockSpec((1, TILE, d), lambda bi, i: (bi, i, 0)),
            pl.BlockSpec((NUM_EXPERTS, d), lambda bi, i: (0, 0)),
            pl.BlockSpec((1, NUM_EXPERTS), lambda bi, i: (0, 0)),
        ],
        out_specs=[ospec, ospec, ospec, ospec],
        out_shape=[flat, flat, flati, flati],
        compiler_params=pltpu.CompilerParams(
            dimension_semantics=("parallel", "parallel"),
        ),
    )(x, W, b2d)
    rw = jnp.stack([w1[:, 0], w2[:, 0]], axis=-1)
    se = jnp.stack([i1[:, 0], i2[:, 0]], axis=-1)
    return rw, se


def kernel(x, W, b):
    b2d = b.reshape(1, NUM_EXPERTS)
    return _run(x, W, b2d)


# speedup vs baseline: 1.5375x; 1.5375x over previous
import functools

import jax
import jax.numpy as jnp
from jax import lax
from jax.experimental import pallas as pl
from jax.experimental.pallas import tpu as pltpu

NUM_EXPERTS = 64
TILE = 1024
NEG_INF = float("-inf")


def _router_kernel(x_ref, w_ref, b_ref, out_ref):
    logits_t = lax.dot_general(
        w_ref[...], x_ref[0],
        dimension_numbers=(((1,), (1,)), ((), ())),
        preferred_element_type=jnp.float32,
    )
    logits_t = logits_t + b_ref[...]

    t = logits_t.shape[1]
    iota = lax.broadcasted_iota(jnp.int32, (NUM_EXPERTS, t), 0)
    big = jnp.int32(NUM_EXPERTS)

    m1 = jnp.max(logits_t, axis=0, keepdims=True)
    i1 = jnp.min(jnp.where(logits_t == m1, iota, big), axis=0, keepdims=True)
    masked = jnp.where(iota == i1, NEG_INF, logits_t)
    m2 = jnp.max(masked, axis=0, keepdims=True)
    i2 = jnp.min(jnp.where(masked == m2, iota, big), axis=0, keepdims=True)

    w1 = jax.nn.sigmoid(m1 - m2)
    w2 = 1.0 - w1

    zeros = jnp.zeros((4, t), jnp.float32)
    out_ref[...] = jnp.concatenate(
        [w1, w2, i1.astype(jnp.float32), i2.astype(jnp.float32), zeros],
        axis=0,
    )


@functools.partial(jax.jit, static_argnames=())
def _run(x, W, bcol):
    bsz, seq, d = x.shape
    nblk = seq // TILE
    grid = (bsz, nblk)
    out = pl.pallas_call(
        _router_kernel,
        grid=grid,
        in_specs=[
            pl.BlockSpec((1, TILE, d), lambda bi, i: (bi, i, 0)),
            pl.BlockSpec((NUM_EXPERTS, d), lambda bi, i: (0, 0)),
            pl.BlockSpec((NUM_EXPERTS, 1), lambda bi, i: (0, 0)),
        ],
        out_specs=pl.BlockSpec(
            (8, TILE), lambda bi, i: (0, bi * nblk + i)
        ),
        out_shape=jax.ShapeDtypeStruct((8, bsz * seq), jnp.float32),
        compiler_params=pltpu.CompilerParams(
            dimension_semantics=("parallel", "parallel"),
        ),
    )(x, W, bcol)
    rw = jnp.stack([out[0], out[1]], axis=-1).reshape(bsz, seq, 2)
    se = jnp.stack([out[2], out[3]], axis=-1).astype(jnp.int32).reshape(bsz, seq, 2)
    return rw, se


def kernel(x, W, b):
    bcol = b.reshape(NUM_EXPERTS, 1)
    return _run(x, W, bcol)
